# baseline (speedup 1.0000x reference)
import functools

import jax
import jax.numpy as jnp
from jax import lax
from jax.experimental import pallas as pl
from jax.experimental.pallas import tpu as pltpu
from jax.experimental.pallas import tpu_sc as plsc

_INFO = plsc.get_sparse_core_info()
_NC, _NS = _INFO.num_cores, _INFO.num_subcores
_NW = _NC * _NS

_B = 4096
_H = 50
_HP = 56
_D = 64
_DP = 128
_N = _B * _H
_BPW = _N // _NW
_NB = 16
_C = _NB * _H
_NCH = _BPW // _C


def _make_lookup():
  mesh = plsc.VectorSubcoreMesh(core_axis_name="c", subcore_axis_name="s")

  @functools.partial(
      pl.kernel,
      out_type=jax.ShapeDtypeStruct((_B, _HP, _DP), jnp.float32),
      mesh=mesh,
      scratch_types=[
          pltpu.VMEM((_BPW // _H, _HP), jnp.int32),
          pltpu.VMEM((_NB * _HP, _D), jnp.float32),
          pltpu.VMEM((_NB * _HP, _D), jnp.float32),
          pltpu.SemaphoreType.DMA,
          pltpu.SemaphoreType.DMA,
          pltpu.SemaphoreType.DMA,
          pltpu.SemaphoreType.DMA,
      ],
      compiler_params=pltpu.CompilerParams(use_tc_tiling_on_sc=False),
  )
  def lookup(table_hbm, idx_hbm, out_hbm, idx_v, rows0, rows1, g0, g1, p0, p1):
    wid = lax.axis_index("s") * _NC + lax.axis_index("c")
    bbase = wid * (_BPW // _H)
    rows = (rows0, rows1)
    gsem = (g0, g1)
    psem = (p0, p1)

    pltpu.sync_copy(
        idx_hbm.at[pl.ds(bbase, _BPW // _H), pl.ds(0, _HP)], idx_v)

    def gather(j, rbuf, gs):
      return [
          pltpu.async_copy(
              table_hbm.at[idx_v.at[j * _NB + k]],
              rbuf.at[pl.ds(k * _HP, _HP)], gs)
          for k in range(_NB)
      ]

    def put(j, rbuf, ps):
      return [
          pltpu.async_copy(
              rbuf.at[pl.ds(k * _HP, _H)],
              out_hbm.at[bbase + j * _NB + k, pl.ds(0, _H), pl.ds(0, _D)],
              ps)
          for k in range(_NB)
      ]

    gets = [None, None]
    puts = [None, None]
    gets[0] = gather(0, rows[0], gsem[0])
    for j in range(1, _NCH):
      b = j % 2
      if puts[b] is not None:
        for c in puts[b]:
          c.wait()
      gets[b] = gather(j, rows[b], gsem[b])
      pb = (j - 1) % 2
      for c in gets[pb]:
        c.wait()
      puts[pb] = put(j - 1, rows[pb], psem[pb])
    lb = (_NCH - 1) % 2
    for c in gets[lb]:
      c.wait()
    puts[lb] = put(_NCH - 1, rows[lb], psem[lb])
    for c in puts[1 - lb]:
      c.wait()
    for c in puts[lb]:
      c.wait()

  return lookup


_LOOKUP = _make_lookup()


@jax.jit
def kernel(input, mask_idx, emb_table):
  del input
  idx = jnp.pad(mask_idx.astype(jnp.int32), ((0, 0), (0, _DP - _H)))
  padded = _LOOKUP(emb_table, idx)
  return lax.slice(padded, (0, 0, 0), (_B, _H, _D))


# speedup vs baseline: 2.5729x; 2.5729x over previous
import functools

import jax
import jax.numpy as jnp
from jax import lax
from jax.experimental import pallas as pl
from jax.experimental.pallas import tpu as pltpu
from jax.experimental.pallas import tpu_sc as plsc

_INFO = plsc.get_sparse_core_info()
_NC, _NS = _INFO.num_cores, _INFO.num_subcores
_NW = _NC * _NS

_B = 4096
_H = 50
_HP = 50
_D = 64
_DP = 128
_N = _B * _H
_BPW = _N // _NW
_NB = 16
_C = _NB * _H
_NCH = _BPW // _C


def _make_lookup():
  mesh = plsc.VectorSubcoreMesh(core_axis_name="c", subcore_axis_name="s")

  @functools.partial(
      pl.kernel,
      out_type=jax.ShapeDtypeStruct((_B, _HP, _DP), jnp.float32),
      mesh=mesh,
      scratch_types=[
          pltpu.VMEM((_NCH, _C), jnp.int32),
          pltpu.VMEM((_C, _D), jnp.float32),
          pltpu.VMEM((_C, _D), jnp.float32),
          pltpu.SemaphoreType.DMA,
          pltpu.SemaphoreType.DMA,
          pltpu.SemaphoreType.DMA,
          pltpu.SemaphoreType.DMA,
      ],
      compiler_params=pltpu.CompilerParams(use_tc_tiling_on_sc=False),
  )
  def lookup(table_hbm, idx_hbm, out_hbm, idx_v, rows0, rows1, g0, g1, p0, p1):
    wid = lax.axis_index("s") * _NC + lax.axis_index("c")
    bbase = wid * (_BPW // _H)
    rows = (rows0, rows1)
    gsem = (g0, g1)
    psem = (p0, p1)

    pltpu.sync_copy(idx_hbm.at[wid], idx_v)

    def gather(j, rbuf, gs):
      return pltpu.async_copy(table_hbm.at[idx_v.at[j]], rbuf, gs)

    def put(j, rbuf, ps):
      return [
          pltpu.async_copy(
              rbuf.at[pl.ds(k * _H, _H)],
              out_hbm.at[bbase + j * _NB + k, pl.ds(0, _H), pl.ds(0, _D)],
              ps)
          for k in range(_NB)
      ]

    gets = [None, None]
    puts = [None, None]
    gets[0] = gather(0, rows[0], gsem[0])
    for j in range(1, _NCH):
      b = j % 2
      if puts[b] is not None:
        for c in puts[b]:
          c.wait()
      gets[b] = gather(j, rows[b], gsem[b])
      pb = (j - 1) % 2
      gets[pb].wait()
      puts[pb] = put(j - 1, rows[pb], psem[pb])
    lb = (_NCH - 1) % 2
    gets[lb].wait()
    puts[lb] = put(_NCH - 1, rows[lb], psem[lb])
    for c in puts[1 - lb]:
      c.wait()
    for c in puts[lb]:
      c.wait()

  return lookup


_LOOKUP = _make_lookup()


@jax.jit
def kernel(input, mask_idx, emb_table):
  del input
  idx = mask_idx.astype(jnp.int32).reshape(_NW, _NCH, _C)
  padded = _LOOKUP(emb_table, idx)
  return lax.slice(padded, (0, 0, 0), (_B, _H, _D))


# speedup vs baseline: 4.2092x; 1.6360x over previous
import functools

import jax
import jax.numpy as jnp
from jax import lax
from jax.experimental import pallas as pl
from jax.experimental.pallas import tpu as pltpu
from jax.experimental.pallas import tpu_sc as plsc

_INFO = plsc.get_sparse_core_info()
_NC, _NS = _INFO.num_cores, _INFO.num_subcores
_NW = _NC * _NS

_B = 4096
_H = 50
_HP = 56
_D = 64
_DP = 128
_N = _B * _H
_BPW = _N // _NW
_NB = 8
_C = _NB * _H
_NCH = _BPW // _C
_NBUF = 4


def _make_lookup():
  mesh = plsc.VectorSubcoreMesh(core_axis_name="c", subcore_axis_name="s")

  @functools.partial(
      pl.kernel,
      out_type=jax.ShapeDtypeStruct((_B, _HP, _DP), jnp.float32),
      mesh=mesh,
      scratch_types=(
          [pltpu.VMEM((_NCH, _C), jnp.int32)]
          + [pltpu.VMEM((_C, _D), jnp.float32)] * _NBUF
          + [pltpu.SemaphoreType.DMA] * (2 * _NBUF)
      ),
      compiler_params=pltpu.CompilerParams(use_tc_tiling_on_sc=False),
  )
  def lookup(table_hbm, idx_hbm, out_hbm, idx_v, *bufs_and_sems):
    rows = bufs_and_sems[:_NBUF]
    gsem = bufs_and_sems[_NBUF:2 * _NBUF]
    psem = bufs_and_sems[2 * _NBUF:]
    wid = lax.axis_index("s") * _NC + lax.axis_index("c")
    bbase = wid * (_BPW // _H)

    pltpu.sync_copy(idx_hbm.at[wid], idx_v)

    def gather(j, rbuf, gs):
      return pltpu.async_copy(table_hbm.at[idx_v.at[j]], rbuf, gs)

    def put(j, rbuf, ps):
      return [
          pltpu.async_copy(
              rbuf.at[pl.ds(k * _H, _H)],
              out_hbm.at[bbase + j * _NB + k, pl.ds(0, _H), pl.ds(0, _D)],
              ps)
          for k in range(_NB)
      ]

    gets = [None] * _NBUF
    puts = [None] * _NBUF
    for j in range(_NBUF - 1):
      gets[j] = gather(j, rows[j], gsem[j])
    for j in range(_NBUF - 1, _NCH):
      b = j % _NBUF
      if puts[b] is not None:
        for c in puts[b]:
          c.wait()
      gets[b] = gather(j, rows[b], gsem[b])
      pb = (j - _NBUF + 1) % _NBUF
      gets[pb].wait()
      puts[pb] = put(j - _NBUF + 1, rows[pb], psem[pb])
    for j in range(_NCH - _NBUF + 1, _NCH):
      b = j % _NBUF
      gets[b].wait()
      puts[b] = put(j, rows[b], psem[b])
    for pt in puts:
      for c in pt:
        c.wait()

  return lookup


_LOOKUP = _make_lookup()


@jax.jit
def kernel(input, mask_idx, emb_table):
  del input
  idx = mask_idx.astype(jnp.int32).reshape(_NW, _NCH, _C)
  padded = _LOOKUP(emb_table, idx)
  return lax.slice(padded, (0, 0, 0), (_B, _H, _D))
